# baseline (speedup 1.0000x reference)
import functools

import jax
import jax.numpy as jnp
from jax import lax
from jax.experimental import pallas as pl
from jax.experimental.pallas import tpu as pltpu
from jax.experimental.pallas import tpu_sc as plsc

VOCAB = 100000
DIM = 64
B = 4096
W = 20

NC = 2
NS = 16
NW = NC * NS

G = 128
IPW = B // NW
OPW = (B * W) // NW
NG = OPW // G
NSLICE = DIM // 16


def _body(ivectors, ovectors, iw, ow, nw, out_i, out_o, out_n,
          idx_i, idx_o, idx_n, rows, sem):
    wid = lax.axis_index("s") * NC + lax.axis_index("c")

    pltpu.sync_copy(iw.at[wid], idx_i)
    pltpu.sync_copy(ow.at[wid], idx_o)
    pltpu.sync_copy(nw.at[wid], idx_n)

    base_i = pl.multiple_of(wid * IPW, IPW)
    pltpu.async_copy(ivectors.at[idx_i.at[0]], rows, sem).wait()
    pltpu.sync_copy(rows, out_i.at[pl.ds(base_i, G)])

    base_o = wid * OPW

    def o_step(j, carry):
        dst = pl.multiple_of(base_o + j * G, G)
        pltpu.async_copy(ovectors.at[idx_o.at[j]], rows, sem).wait()
        pltpu.sync_copy(rows, out_o.at[pl.ds(dst, G)])
        return carry

    lax.fori_loop(0, NG, o_step, 0)

    def n_step(j, carry):
        dst = pl.multiple_of(base_o + j * G, G)
        pltpu.async_copy(ovectors.at[idx_n.at[j]], rows, sem).wait()

        def neg_row(r, c):
            for col in range(NSLICE):
                s = pl.ds(col * 16, 16)
                rows[r, s] = -rows[r, s]
            return c

        lax.fori_loop(0, G, neg_row, 0)
        pltpu.sync_copy(rows, out_n.at[pl.ds(dst, G)])
        return carry

    lax.fori_loop(0, NG, n_step, 0)


@jax.jit
def kernel(iwords, owords, nwords, ivectors, ovectors):
    iw = iwords.astype(jnp.int32).reshape(NW, 1, IPW)
    ow = owords.astype(jnp.int32).reshape(NW, NG, G)
    nw = nwords.astype(jnp.int32).reshape(NW, NG, G)

    mesh = plsc.VectorSubcoreMesh(core_axis_name="c", subcore_axis_name="s")
    out_i, out_o, out_n = pl.kernel(
        _body,
        out_type=(
            jax.ShapeDtypeStruct((B, DIM), jnp.float32),
            jax.ShapeDtypeStruct((B * W, DIM), jnp.float32),
            jax.ShapeDtypeStruct((B * W, DIM), jnp.float32),
        ),
        mesh=mesh,
        compiler_params=pltpu.CompilerParams(use_tc_tiling_on_sc=False),
        scratch_types=[
            pltpu.VMEM((1, IPW), jnp.int32),
            pltpu.VMEM((NG, G), jnp.int32),
            pltpu.VMEM((NG, G), jnp.int32),
            pltpu.VMEM((G, DIM), jnp.float32),
            pltpu.SemaphoreType.DMA,
        ],
    )(ivectors, ovectors, iw, ow, nw)

    return (out_i,
            out_o.reshape(B, W, DIM),
            (out_n.reshape(B, W, DIM)))


# speedup vs baseline: 1.1479x; 1.1479x over previous
import jax
import jax.numpy as jnp
from jax import lax
from jax.experimental import pallas as pl
from jax.experimental.pallas import tpu as pltpu
from jax.experimental.pallas import tpu_sc as plsc

VOCAB = 100000
DIM = 64
B = 4096
W = 20

NC = 2
NS = 16
NW = NC * NS

G = 128
IPW = B // NW
OPW = (B * W) // NW
NG = OPW // G
GPC = 5
CR = G * GPC
NSC = NG // GPC
NSLICE = DIM // 16


def _body(ivectors, ovectors, iw, ow, nw, out_i, out_o, out_n,
          idx_i, idx_o, idx_n, buf0, buf1, sem0, sem1, semi):
    wid = lax.axis_index("s") * NC + lax.axis_index("c")

    pltpu.sync_copy(iw.at[wid], idx_i)
    pltpu.sync_copy(ow.at[wid], idx_o)
    pltpu.sync_copy(nw.at[wid], idx_n)

    bufs = (buf0, buf1)
    sems = (sem0, sem1)
    base_o = wid * OPW

    work = ([(idx_o, out_o, sc, False) for sc in range(NSC)]
            + [(idx_n, out_n, sc, True) for sc in range(NSC)])

    def issue(item, slot):
        idx, _, sc, _ = item
        return [
            pltpu.async_copy(
                ovectors.at[idx.at[sc * GPC + k]],
                bufs[slot].at[pl.ds(k * G, G)],
                sems[slot],
            )
            for k in range(GPC)
        ]

    handles = issue(work[0], 0)
    ih = pltpu.async_copy(ivectors.at[idx_i.at[0]], buf1.at[pl.ds(0, G)], semi)

    ih.wait()
    pltpu.sync_copy(buf1.at[pl.ds(0, G)], out_i.at[pl.ds(wid * IPW, G)])

    for i, item in enumerate(work):
        cur = i % 2
        if i + 1 < len(work):
            nxt_handles = issue(work[i + 1], 1 - cur)
        for h in handles:
            h.wait()
        _, out, sc, negate = item
        buf = bufs[cur]
        if negate:
            def neg_rows(r, c):
                for rr in range(4):
                    for col in range(NSLICE):
                        s = pl.ds(col * 16, 16)
                        buf[r * 4 + rr, s] = -buf[r * 4 + rr, s]
                return c

            lax.fori_loop(0, CR // 4, neg_rows, 0)
        pltpu.sync_copy(buf, out.at[pl.ds(base_o + sc * CR, CR)])
        if i + 1 < len(work):
            handles = nxt_handles


@jax.jit
def kernel(iwords, owords, nwords, ivectors, ovectors):
    iw = iwords.astype(jnp.int32).reshape(NW, 1, IPW)
    ow = owords.astype(jnp.int32).reshape(NW, NG, G)
    nw = nwords.astype(jnp.int32).reshape(NW, NG, G)

    mesh = plsc.VectorSubcoreMesh(core_axis_name="c", subcore_axis_name="s")
    out_i, out_o, out_n = pl.kernel(
        _body,
        out_type=(
            jax.ShapeDtypeStruct((B, DIM), jnp.float32),
            jax.ShapeDtypeStruct((B * W, DIM), jnp.float32),
            jax.ShapeDtypeStruct((B * W, DIM), jnp.float32),
        ),
        mesh=mesh,
        compiler_params=pltpu.CompilerParams(use_tc_tiling_on_sc=False),
        scratch_types=[
            pltpu.VMEM((1, IPW), jnp.int32),
            pltpu.VMEM((NG, G), jnp.int32),
            pltpu.VMEM((NG, G), jnp.int32),
            pltpu.VMEM((CR, DIM), jnp.float32),
            pltpu.VMEM((CR, DIM), jnp.float32),
            pltpu.SemaphoreType.DMA,
            pltpu.SemaphoreType.DMA,
            pltpu.SemaphoreType.DMA,
        ],
    )(ivectors, ovectors, iw, ow, nw)

    return (out_i,
            out_o.reshape(B, W, DIM),
            out_n.reshape(B, W, DIM))
